# baseline (speedup 1.0000x reference)
import functools

import jax
import jax.numpy as jnp
from jax import lax
from jax.experimental import pallas as pl
from jax.experimental.pallas import tpu as pltpu
from jax.experimental.pallas import tpu_sc as plsc

_LANES = 16
_NUM_WORKERS = 32
_CHUNK = 128


@functools.lru_cache(maxsize=None)
def _make_sc_kernel(batch: int, dim: int):
    b_per_w = batch // _NUM_WORKERS
    assert batch % (_NUM_WORKERS * _LANES) == 0
    assert b_per_w % _CHUNK == 0
    assert dim == 2 * _LANES
    n_chunks = b_per_w // _CHUNK

    mesh = plsc.VectorSubcoreMesh(core_axis_name="c", subcore_axis_name="s")

    @functools.partial(
        pl.kernel,
        mesh=mesh,
        compiler_params=pltpu.CompilerParams(needs_layout_passes=False),
        out_type=jax.ShapeDtypeStruct((batch,), jnp.float32),
        scratch_types=[
            pltpu.VMEM((b_per_w,), jnp.int32),
            pltpu.VMEM((b_per_w,), jnp.int32),
            pltpu.VMEM((_CHUNK, dim), jnp.float32),
            pltpu.VMEM((_CHUNK, dim), jnp.float32),
            pltpu.VMEM((_CHUNK, 1), jnp.float32),
            pltpu.VMEM((_CHUNK, 1), jnp.float32),
            pltpu.VMEM((b_per_w,), jnp.float32),
            pltpu.SemaphoreType.DMA,
        ],
    )
    def k(uidx_hbm, midx_hbm, uemb_hbm, memb_hbm, ub_hbm, mb_hbm, out_hbm,
          uidx_v, midx_v, urows_v, mrows_v, ub_v, mb_v,
          out_v, sem):
        wid = lax.axis_index("s") * 2 + lax.axis_index("c")
        base = wid * b_per_w
        pltpu.sync_copy(uidx_hbm.at[pl.ds(base, b_per_w)], uidx_v)
        pltpu.sync_copy(midx_hbm.at[pl.ds(base, b_per_w)], midx_v)

        zeros = jnp.zeros((_LANES,), jnp.int32)
        lane = lax.iota(jnp.int32, _LANES)

        def chunk_body(c, carry):
            cbase = c * _CHUNK

            def fetch(g, fcarry):
                f0 = g * _LANES
                uvec = uidx_v[pl.ds(cbase + f0, _LANES)]
                mvec = midx_v[pl.ds(cbase + f0, _LANES)]
                for r in range(_LANES):
                    i = f0 + r
                    ui = uvec[r]
                    mi = mvec[r]
                    pltpu.async_copy(uemb_hbm.at[pl.ds(ui, 1), :],
                                     urows_v.at[pl.ds(i, 1), :], sem)
                    pltpu.async_copy(memb_hbm.at[pl.ds(mi, 1), :],
                                     mrows_v.at[pl.ds(i, 1), :], sem)
                    pltpu.async_copy(ub_hbm.at[pl.ds(ui, 1), :],
                                     ub_v.at[pl.ds(i, 1), :], sem)
                    pltpu.async_copy(mb_hbm.at[pl.ds(mi, 1), :],
                                     mb_v.at[pl.ds(i, 1), :], sem)
                return fcarry

            lax.fori_loop(0, _CHUNK // _LANES, fetch, 0)
            pltpu.make_async_copy(uemb_hbm.at[pl.ds(0, _CHUNK), :], urows_v,
                                  sem).wait()
            pltpu.make_async_copy(memb_hbm.at[pl.ds(0, _CHUNK), :], mrows_v,
                                  sem).wait()
            pltpu.make_async_copy(ub_hbm.at[pl.ds(0, _CHUNK), :], ub_v,
                                  sem).wait()
            pltpu.make_async_copy(mb_hbm.at[pl.ds(0, _CHUNK), :], mb_v,
                                  sem).wait()

            def group(g, gcarry):
                b0 = g * _LANES
                rowi = lane + b0
                out_v[pl.ds(cbase + b0, _LANES)] = (
                    plsc.load_gather(ub_v, [rowi, zeros])
                    + plsc.load_gather(mb_v, [rowi, zeros]))
                for r in range(_LANES):
                    b = b0 + r
                    t = (urows_v[b, pl.ds(0, _LANES)]
                         * mrows_v[b, pl.ds(0, _LANES)]
                         + urows_v[b, pl.ds(_LANES, _LANES)]
                         * mrows_v[b, pl.ds(_LANES, _LANES)])
                    plsc.addupdate_scatter(out_v, [zeros + cbase + b], t)
                return gcarry

            lax.fori_loop(0, _CHUNK // _LANES, group, 0)
            return carry

        lax.fori_loop(0, n_chunks, chunk_body, 0)
        pltpu.sync_copy(out_v, out_hbm.at[pl.ds(base, b_per_w)])

    return k


def kernel(user_indices, movie_indices, user_emb, movie_emb, user_bias, movie_bias):
    batch = user_indices.shape[0]
    dim = user_emb.shape[1]
    k = _make_sc_kernel(batch, dim)
    return k(
        user_indices.astype(jnp.int32),
        movie_indices.astype(jnp.int32),
        user_emb,
        movie_emb,
        user_bias,
        movie_bias,
    )


# speedup vs baseline: 2.5871x; 2.5871x over previous
import functools

import jax
import jax.numpy as jnp
from jax import lax
from jax.experimental import pallas as pl
from jax.experimental.pallas import tpu as pltpu
from jax.experimental.pallas import tpu_sc as plsc

_L = 16
_NW = 32
_BLK = 1024
_CAP = 4096
_FCAP = 128
_STAGE_W = 128


def _owned_blocks(w, nblk_total):
    base = nblk_total // _NW
    extra = nblk_total % _NW
    nblk = base + jnp.where(w < extra, 1, 0)
    blk_lo = w * base + jnp.minimum(w, extra)
    return blk_lo, nblk


@functools.lru_cache(maxsize=None)
def _make_phase1(batch: int, dim: int, nu: int, nm: int):
    mesh = plsc.VectorSubcoreMesh(core_axis_name="c", subcore_axis_name="s")
    nblk_u = -(-nu // _BLK)
    nblk_m = -(-nm // _BLK)

    @functools.partial(
        pl.kernel,
        mesh=mesh,
        compiler_params=pltpu.CompilerParams(needs_layout_passes=False),
        out_type=(
            jax.ShapeDtypeStruct((batch, _STAGE_W), jnp.float32),
            jax.ShapeDtypeStruct((batch, _STAGE_W), jnp.float32),
        ),
        scratch_types=[
            pltpu.VMEM((batch,), jnp.int32),
            pltpu.VMEM((_CAP,), jnp.int32),
            pltpu.VMEM((_CAP,), jnp.int32),
            pltpu.VMEM((_CAP,), jnp.int32),
            pltpu.VMEM((_CAP,), jnp.int32),
            pltpu.VMEM((dim, _BLK), jnp.float32),
            pltpu.VMEM((_BLK,), jnp.float32),
            pltpu.VMEM((_FCAP, _STAGE_W), jnp.float32),
            pltpu.VMEM((_FCAP,), jnp.int32),
            pltpu.SemaphoreType.DMA,
            pltpu.SemaphoreType.DMA,
        ],
    )
    def k(uidx_hbm, midx_hbm, uembt_hbm, membt_hbm, ub_hbm, mb_hbm,
          stage_u_hbm, stage_m_hbm,
          idx_v, own_rel, own_b, hit_rel, hit_b, blk0, bias0,
          rowflight, flight_b, sem, ssem):
        w = lax.axis_index("s") * 2 + lax.axis_index("c")
        lane = lax.iota(jnp.int32, _L)
        zeros = jnp.zeros((_L,), jnp.int32)
        neg1 = zeros - 1

        def reset_flight_b():
            for q in range(_FCAP // _L):
                flight_b[pl.ds(q * _L, _L)] = neg1

        def run_side(idx_hbm, embt_hbm, b_hbm, stage_hbm, nblk_total, nrows):
            blk_lo, nblk = _owned_blocks(w, nblk_total)
            row_lo = blk_lo * _BLK
            pltpu.sync_copy(idx_hbm, idx_v)
            nrows_my = nblk * _BLK

            def sel(g, ptr):
                v = idx_v[pl.ds(g * _L, _L)]
                rel = v - row_lo
                msk = (rel >= 0) & (rel < nrows_my)
                p = jnp.minimum(ptr, _CAP - _L)
                plsc.store_compressed(own_rel.at[pl.ds(p, _L)], rel, mask=msk)
                plsc.store_compressed(own_b.at[pl.ds(p, _L)],
                                      lane + g * _L, mask=msk)
                cnt = plsc.all_reduce_population_count(msk)
                return ptr + cnt[0]

            count = lax.fori_loop(0, batch // _L, sel, 0)
            count = jnp.minimum(count, _CAP)

            reset_flight_b()

            def block_body(blk, carry):
                fslot = carry
                col0 = pl.multiple_of((blk_lo + blk) * _BLK, _BLK)
                pltpu.async_copy(embt_hbm.at[:, pl.ds(col0, _BLK)],
                                 blk0, sem)
                pltpu.async_copy(b_hbm.at[pl.ds(col0, _BLK)], bias0, sem)
                pltpu.make_async_copy(
                    embt_hbm.at[:, pl.ds(0, _BLK)], blk0, sem).wait()
                pltpu.make_async_copy(
                    b_hbm.at[pl.ds(0, _BLK)], bias0, sem).wait()
                lo = blk * _BLK

                def collect(g, hptr):
                    gp = g * _L
                    rel = own_rel[pl.ds(jnp.minimum(gp, _CAP - _L), _L)]
                    bb = own_b[pl.ds(jnp.minimum(gp, _CAP - _L), _L)]
                    msk = ((gp + lane < count)
                           & (rel >= lo) & (rel < lo + _BLK))
                    p = jnp.minimum(hptr, _CAP - _L)
                    plsc.store_compressed(hit_rel.at[pl.ds(p, _L)],
                                          rel - lo, mask=msk)
                    plsc.store_compressed(hit_b.at[pl.ds(p, _L)], bb, mask=msk)
                    cnt = plsc.all_reduce_population_count(msk)
                    return hptr + cnt[0]

                hcount = lax.fori_loop(0, (count + _L - 1) // _L, collect, 0)

                def extract(h, fs):
                    hp = h * _L
                    rel = hit_rel[pl.ds(jnp.minimum(hp, _CAP - _L), _L)]
                    bb = hit_b[pl.ds(jnp.minimum(hp, _CAP - _L), _L)]
                    valid = hp + lane < hcount
                    nv = plsc.all_reduce_population_count(valid)[0]
                    slots = fs + lane
                    for d in range(dim):
                        comp = plsc.load_gather(blk0, [zeros + d, rel],
                                                mask=valid)
                        plsc.store_scatter(rowflight, [slots, zeros + d],
                                           comp, mask=valid)
                    bv = plsc.load_gather(bias0, [rel], mask=valid)
                    plsc.store_scatter(rowflight, [slots, zeros + dim],
                                       bv, mask=valid)
                    plsc.store_scatter(flight_b, [slots],
                                       jnp.where(valid, bb, neg1), mask=valid)
                    fs = fs + nv

                    @pl.when(fs > _FCAP - _L)
                    def _():
                        pltpu.async_copy(
                            rowflight,
                            stage_hbm.at[plsc.Indices(flight_b,
                                                      ignored_value=-1)],
                            ssem).wait()
                        reset_flight_b()

                    return jnp.where(fs > _FCAP - _L, 0, fs)

                fslot = lax.fori_loop(
                    0, (hcount + _L - 1) // _L, extract, fslot)
                return fslot

            fslot = lax.fori_loop(0, nblk, block_body, 0)

            @pl.when(fslot > 0)
            def _():
                pltpu.async_copy(
                    rowflight,
                    stage_hbm.at[plsc.Indices(flight_b, ignored_value=-1)],
                    ssem).wait()

        run_side(uidx_hbm, uembt_hbm, ub_hbm, stage_u_hbm, nblk_u, nu)
        run_side(midx_hbm, membt_hbm, mb_hbm, stage_m_hbm, nblk_m, nm)

    return k


@functools.lru_cache(maxsize=None)
def _make_phase2(batch: int, dim: int):
    mesh = plsc.VectorSubcoreMesh(core_axis_name="c", subcore_axis_name="s")
    b_per_w = batch // _NW
    chunk = 128
    n_chunks = b_per_w // chunk

    @functools.partial(
        pl.kernel,
        mesh=mesh,
        compiler_params=pltpu.CompilerParams(needs_layout_passes=False),
        out_type=jax.ShapeDtypeStruct((batch,), jnp.float32),
        scratch_types=[
            pltpu.VMEM((chunk, _STAGE_W), jnp.float32),
            pltpu.VMEM((chunk, _STAGE_W), jnp.float32),
            pltpu.VMEM((b_per_w,), jnp.float32),
            pltpu.SemaphoreType.DMA,
        ],
    )
    def k(stage_u_hbm, stage_m_hbm, out_hbm, cu, cm, out_v, sem):
        w = lax.axis_index("s") * 2 + lax.axis_index("c")
        base = w * b_per_w
        lane = lax.iota(jnp.int32, _L)
        zeros = jnp.zeros((_L,), jnp.int32)

        def chunk_body(c, carry):
            r0 = base + c * chunk
            pltpu.async_copy(stage_u_hbm.at[pl.ds(r0, chunk), :], cu, sem)
            pltpu.async_copy(stage_m_hbm.at[pl.ds(r0, chunk), :], cm, sem)
            pltpu.make_async_copy(stage_u_hbm.at[pl.ds(r0, chunk), :], cu,
                                  sem).wait()
            pltpu.make_async_copy(stage_m_hbm.at[pl.ds(r0, chunk), :], cm,
                                  sem).wait()
            o0 = c * chunk

            def group(g, gcarry):
                b0 = g * _L
                rows = lane + b0
                out_v[pl.ds(o0 + b0, _L)] = (
                    plsc.load_gather(cu, [rows, zeros + dim])
                    + plsc.load_gather(cm, [rows, zeros + dim]))
                for r in range(_L):
                    b = b0 + r
                    t = (cu[b, pl.ds(0, _L)] * cm[b, pl.ds(0, _L)]
                         + cu[b, pl.ds(_L, _L)] * cm[b, pl.ds(_L, _L)])
                    plsc.addupdate_scatter(out_v, [zeros + o0 + b], t)
                return gcarry

            lax.fori_loop(0, chunk // _L, group, 0)
            return carry

        lax.fori_loop(0, n_chunks, chunk_body, 0)
        pltpu.sync_copy(out_v, out_hbm.at[pl.ds(base, b_per_w)])

    return k


def kernel(user_indices, movie_indices, user_emb, movie_emb, user_bias, movie_bias):
    batch = user_indices.shape[0]
    nu, dim = user_emb.shape
    nm = movie_emb.shape[0]
    p1 = _make_phase1(batch, dim, nu, nm)
    p2 = _make_phase2(batch, dim)
    stage_u, stage_m = p1(
        user_indices.astype(jnp.int32),
        movie_indices.astype(jnp.int32),
        user_emb.T,
        movie_emb.T,
        user_bias.reshape(-1),
        movie_bias.reshape(-1),
    )
    return p2(stage_u, stage_m)


# speedup vs baseline: 2.9640x; 1.1457x over previous
import functools

import jax
import jax.numpy as jnp
from jax import lax
from jax.experimental import pallas as pl
from jax.experimental.pallas import tpu as pltpu
from jax.experimental.pallas import tpu_sc as plsc

_L = 16
_NW = 32
_BLK = 1024
_CAP = 4096
_FCAP = 128
_STAGE_W = 128


def _owned_blocks(w, nblk_total):
    base = nblk_total // _NW
    extra = nblk_total % _NW
    nblk = base + jnp.where(w < extra, 1, 0)
    blk_lo = w * base + jnp.minimum(w, extra)
    return blk_lo, nblk


@functools.lru_cache(maxsize=None)
def _make_phase1(batch: int, dim: int, nu: int, nm: int):
    mesh = plsc.VectorSubcoreMesh(core_axis_name="c", subcore_axis_name="s")
    nblk_u = -(-nu // _BLK)
    nblk_m = -(-nm // _BLK)

    @functools.partial(
        pl.kernel,
        mesh=mesh,
        compiler_params=pltpu.CompilerParams(needs_layout_passes=False),
        out_type=(
            jax.ShapeDtypeStruct((batch, _STAGE_W), jnp.float32),
            jax.ShapeDtypeStruct((batch, _STAGE_W), jnp.float32),
        ),
        scratch_types=[
            pltpu.VMEM((batch,), jnp.int32),
            pltpu.VMEM((_CAP,), jnp.int32),
            pltpu.VMEM((_CAP,), jnp.int32),
            pltpu.VMEM((_CAP,), jnp.int32),
            pltpu.VMEM((_CAP,), jnp.int32),
            pltpu.VMEM((dim, _BLK), jnp.float32),
            pltpu.VMEM((dim, _BLK), jnp.float32),
            pltpu.VMEM((_BLK,), jnp.float32),
            pltpu.VMEM((_BLK,), jnp.float32),
            pltpu.VMEM((_FCAP, _STAGE_W), jnp.float32),
            pltpu.VMEM((_FCAP,), jnp.int32),
            pltpu.SemaphoreType.DMA,
            pltpu.SemaphoreType.DMA,
            pltpu.SemaphoreType.DMA,
        ],
    )
    def k(uidx_hbm, midx_hbm, uembt_hbm, membt_hbm, ub_hbm, mb_hbm,
          stage_u_hbm, stage_m_hbm,
          idx_v, own_rel, own_b, hit_rel, hit_b, blk0, blk1, bias0, bias1,
          rowflight, flight_b, sem0, sem1, ssem):
        w = lax.axis_index("s") * 2 + lax.axis_index("c")
        lane = lax.iota(jnp.int32, _L)
        zeros = jnp.zeros((_L,), jnp.int32)
        neg1 = zeros - 1

        def reset_flight_b():
            for q in range(_FCAP // _L):
                flight_b[pl.ds(q * _L, _L)] = neg1

        def run_side(idx_hbm, embt_hbm, b_hbm, stage_hbm, nblk_total, nrows):
            blk_lo, nblk = _owned_blocks(w, nblk_total)
            row_lo = blk_lo * _BLK
            pltpu.sync_copy(idx_hbm, idx_v)
            nrows_my = nblk * _BLK

            def sel(g, ptr):
                v = idx_v[pl.ds(g * _L, _L)]
                rel = v - row_lo
                msk = (rel >= 0) & (rel < nrows_my)
                p = jnp.minimum(ptr, _CAP - _L)
                plsc.store_compressed(own_rel.at[pl.ds(p, _L)], rel, mask=msk)
                plsc.store_compressed(own_b.at[pl.ds(p, _L)],
                                      lane + g * _L, mask=msk)
                cnt = plsc.all_reduce_population_count(msk)
                return ptr + cnt[0]

            count = lax.fori_loop(0, batch // _L, sel, 0)
            count = jnp.minimum(count, _CAP)

            reset_flight_b()

            def startb(blk, blkbuf, biasbuf, s):
                bb = jnp.minimum(blk, nblk - 1)
                col0 = pl.multiple_of((blk_lo + bb) * _BLK, _BLK)
                pltpu.async_copy(embt_hbm.at[:, pl.ds(col0, _BLK)], blkbuf, s)
                pltpu.async_copy(b_hbm.at[pl.ds(col0, _BLK)], biasbuf, s)

            def waitb(blkbuf, biasbuf, s):
                pltpu.make_async_copy(
                    embt_hbm.at[:, pl.ds(0, _BLK)], blkbuf, s).wait()
                pltpu.make_async_copy(
                    b_hbm.at[pl.ds(0, _BLK)], biasbuf, s).wait()

            def process(blk, blkbuf, biasbuf, fslot):
                lo = blk * _BLK

                def collect(g, hptr):
                    gp = g * _L
                    rel = own_rel[pl.ds(jnp.minimum(gp, _CAP - _L), _L)]
                    bb = own_b[pl.ds(jnp.minimum(gp, _CAP - _L), _L)]
                    msk = ((gp + lane < count)
                           & (rel >= lo) & (rel < lo + _BLK))
                    p = jnp.minimum(hptr, _CAP - _L)
                    plsc.store_compressed(hit_rel.at[pl.ds(p, _L)],
                                          rel - lo, mask=msk)
                    plsc.store_compressed(hit_b.at[pl.ds(p, _L)], bb, mask=msk)
                    cnt = plsc.all_reduce_population_count(msk)
                    return hptr + cnt[0]

                hcount = lax.fori_loop(0, (count + _L - 1) // _L, collect, 0)

                def extract(h, fs):
                    hp = h * _L
                    rel = hit_rel[pl.ds(jnp.minimum(hp, _CAP - _L), _L)]
                    bb = hit_b[pl.ds(jnp.minimum(hp, _CAP - _L), _L)]
                    valid = hp + lane < hcount
                    nv = plsc.all_reduce_population_count(valid)[0]
                    slots = fs + lane
                    for d in range(dim):
                        comp = plsc.load_gather(blkbuf, [zeros + d, rel],
                                                mask=valid)
                        plsc.store_scatter(rowflight, [slots, zeros + d],
                                           comp, mask=valid)
                    bv = plsc.load_gather(biasbuf, [rel], mask=valid)
                    plsc.store_scatter(rowflight, [slots, zeros + dim],
                                       bv, mask=valid)
                    plsc.store_scatter(flight_b, [slots],
                                       jnp.where(valid, bb, neg1), mask=valid)
                    fs = fs + nv

                    @pl.when(fs > _FCAP - _L)
                    def _():
                        pltpu.async_copy(
                            rowflight,
                            stage_hbm.at[plsc.Indices(flight_b,
                                                      ignored_value=-1)],
                            ssem).wait()
                        reset_flight_b()

                    return jnp.where(fs > _FCAP - _L, 0, fs)

                return lax.fori_loop(
                    0, (hcount + _L - 1) // _L, extract, fslot)

            startb(0, blk0, bias0, sem0)

            def pair_body(p, fslot):
                startb(2 * p + 1, blk1, bias1, sem1)
                waitb(blk0, bias0, sem0)
                fslot = process(jnp.minimum(2 * p, nblk - 1),
                                blk0, bias0, fslot)
                startb(2 * p + 2, blk0, bias0, sem0)
                waitb(blk1, bias1, sem1)
                fslot = process(jnp.minimum(2 * p + 1, nblk - 1),
                                blk1, bias1, fslot)
                return fslot

            fslot = lax.fori_loop(0, (nblk + 1) // 2, pair_body, 0)
            waitb(blk0, bias0, sem0)

            @pl.when(fslot > 0)
            def _():
                pltpu.async_copy(
                    rowflight,
                    stage_hbm.at[plsc.Indices(flight_b, ignored_value=-1)],
                    ssem).wait()

        run_side(uidx_hbm, uembt_hbm, ub_hbm, stage_u_hbm, nblk_u, nu)
        run_side(midx_hbm, membt_hbm, mb_hbm, stage_m_hbm, nblk_m, nm)

    return k


@functools.lru_cache(maxsize=None)
def _make_phase2(batch: int, dim: int):
    mesh = plsc.VectorSubcoreMesh(core_axis_name="c", subcore_axis_name="s")
    b_per_w = batch // _NW
    chunk = 128
    n_chunks = b_per_w // chunk

    @functools.partial(
        pl.kernel,
        mesh=mesh,
        compiler_params=pltpu.CompilerParams(needs_layout_passes=False),
        out_type=jax.ShapeDtypeStruct((batch,), jnp.float32),
        scratch_types=[
            pltpu.VMEM((chunk, _STAGE_W), jnp.float32),
            pltpu.VMEM((chunk, _STAGE_W), jnp.float32),
            pltpu.VMEM((b_per_w,), jnp.float32),
            pltpu.SemaphoreType.DMA,
        ],
    )
    def k(stage_u_hbm, stage_m_hbm, out_hbm, cu, cm, out_v, sem):
        w = lax.axis_index("s") * 2 + lax.axis_index("c")
        base = w * b_per_w
        lane = lax.iota(jnp.int32, _L)
        zeros = jnp.zeros((_L,), jnp.int32)

        def chunk_body(c, carry):
            r0 = base + c * chunk
            pltpu.async_copy(stage_u_hbm.at[pl.ds(r0, chunk), :], cu, sem)
            pltpu.async_copy(stage_m_hbm.at[pl.ds(r0, chunk), :], cm, sem)
            pltpu.make_async_copy(stage_u_hbm.at[pl.ds(r0, chunk), :], cu,
                                  sem).wait()
            pltpu.make_async_copy(stage_m_hbm.at[pl.ds(r0, chunk), :], cm,
                                  sem).wait()
            o0 = c * chunk

            def group(g, gcarry):
                b0 = g * _L
                rows = lane + b0
                out_v[pl.ds(o0 + b0, _L)] = (
                    plsc.load_gather(cu, [rows, zeros + dim])
                    + plsc.load_gather(cm, [rows, zeros + dim]))
                for r in range(_L):
                    b = b0 + r
                    t = (cu[b, pl.ds(0, _L)] * cm[b, pl.ds(0, _L)]
                         + cu[b, pl.ds(_L, _L)] * cm[b, pl.ds(_L, _L)])
                    plsc.addupdate_scatter(out_v, [zeros + o0 + b], t)
                return gcarry

            lax.fori_loop(0, chunk // _L, group, 0)
            return carry

        lax.fori_loop(0, n_chunks, chunk_body, 0)
        pltpu.sync_copy(out_v, out_hbm.at[pl.ds(base, b_per_w)])

    return k


def kernel(user_indices, movie_indices, user_emb, movie_emb, user_bias, movie_bias):
    batch = user_indices.shape[0]
    nu, dim = user_emb.shape
    nm = movie_emb.shape[0]
    p1 = _make_phase1(batch, dim, nu, nm)
    p2 = _make_phase2(batch, dim)
    stage_u, stage_m = p1(
        user_indices.astype(jnp.int32),
        movie_indices.astype(jnp.int32),
        user_emb.T,
        movie_emb.T,
        user_bias.reshape(-1),
        movie_bias.reshape(-1),
    )
    return p2(stage_u, stage_m)


# speedup vs baseline: 3.7623x; 1.2693x over previous
import functools

import jax
import jax.numpy as jnp
from jax import lax
from jax.experimental import pallas as pl
from jax.experimental.pallas import tpu as pltpu
from jax.experimental.pallas import tpu_sc as plsc

_L = 16
_NW = 32
_BLK = 1024
_CAP = 4096
_FCAP = 128
_STAGE_W = 128


def _owned_blocks(w, nblk_total):
    base = nblk_total // _NW
    extra = nblk_total % _NW
    nblk = base + jnp.where(w < extra, 1, 0)
    blk_lo = w * base + jnp.minimum(w, extra)
    return blk_lo, nblk


@functools.lru_cache(maxsize=None)
def _make_phase1(batch: int, dim: int, nu: int, nm: int):
    mesh = plsc.VectorSubcoreMesh(core_axis_name="c", subcore_axis_name="s")
    nblk_u = -(-nu // _BLK)
    nblk_m = -(-nm // _BLK)

    @functools.partial(
        pl.kernel,
        mesh=mesh,
        compiler_params=pltpu.CompilerParams(needs_layout_passes=False),
        out_type=(
            jax.ShapeDtypeStruct((batch, _STAGE_W), jnp.float32),
            jax.ShapeDtypeStruct((batch, _STAGE_W), jnp.float32),
        ),
        scratch_types=[
            pltpu.VMEM((batch,), jnp.int32),
            pltpu.VMEM((_CAP,), jnp.int32),
            pltpu.VMEM((_CAP,), jnp.int32),
            pltpu.VMEM((_CAP,), jnp.int32),
            pltpu.VMEM((_CAP,), jnp.int32),
            pltpu.VMEM((dim, _BLK), jnp.float32),
            pltpu.VMEM((dim, _BLK), jnp.float32),
            pltpu.VMEM((_FCAP, _STAGE_W), jnp.float32),
            pltpu.VMEM((_FCAP,), jnp.int32),
            pltpu.SemaphoreType.DMA,
            pltpu.SemaphoreType.DMA,
            pltpu.SemaphoreType.DMA,
        ],
    )
    def k(uidx_hbm, midx_hbm, uembt_hbm, membt_hbm,
          stage_u_hbm, stage_m_hbm,
          idx_v, own_rel, own_b, hit_rel, hit_b, blk0, blk1,
          rowflight, flight_b, sem0, sem1, ssem):
        w = lax.axis_index("s") * 2 + lax.axis_index("c")
        lane = lax.iota(jnp.int32, _L)
        zeros = jnp.zeros((_L,), jnp.int32)
        neg1 = zeros - 1

        def reset_flight_b():
            for q in range(_FCAP // _L):
                flight_b[pl.ds(q * _L, _L)] = neg1

        def run_side(idx_hbm, embt_hbm, stage_hbm, nblk_total, nrows):
            blk_lo, nblk = _owned_blocks(w, nblk_total)
            row_lo = blk_lo * _BLK
            pltpu.sync_copy(idx_hbm, idx_v)
            nrows_my = nblk * _BLK

            def sel(g, ptr):
                v = idx_v[pl.ds(g * _L, _L)]
                rel = v - row_lo
                msk = (rel >= 0) & (rel < nrows_my)
                p = jnp.minimum(ptr, _CAP - _L)
                plsc.store_compressed(own_rel.at[pl.ds(p, _L)], rel, mask=msk)
                plsc.store_compressed(own_b.at[pl.ds(p, _L)],
                                      lane + g * _L, mask=msk)
                cnt = plsc.all_reduce_population_count(msk)
                return ptr + cnt[0]

            count = lax.fori_loop(0, batch // _L, sel, 0)
            count = jnp.minimum(count, _CAP)

            reset_flight_b()

            def startb(blk, blkbuf, s):
                bb = jnp.minimum(blk, nblk - 1)
                col0 = pl.multiple_of((blk_lo + bb) * _BLK, _BLK)
                pltpu.async_copy(embt_hbm.at[:, pl.ds(col0, _BLK)], blkbuf, s)

            def waitb(blkbuf, s):
                pltpu.make_async_copy(
                    embt_hbm.at[:, pl.ds(0, _BLK)], blkbuf, s).wait()

            def process(blk, blkbuf, fslot):
                lo = blk * _BLK

                def collect(g, hptr):
                    gp = g * _L
                    rel = own_rel[pl.ds(jnp.minimum(gp, _CAP - _L), _L)]
                    bb = own_b[pl.ds(jnp.minimum(gp, _CAP - _L), _L)]
                    msk = ((gp + lane < count)
                           & (rel >= lo) & (rel < lo + _BLK))
                    p = jnp.minimum(hptr, _CAP - _L)
                    plsc.store_compressed(hit_rel.at[pl.ds(p, _L)],
                                          rel - lo, mask=msk)
                    plsc.store_compressed(hit_b.at[pl.ds(p, _L)], bb, mask=msk)
                    cnt = plsc.all_reduce_population_count(msk)
                    return hptr + cnt[0]

                hcount = lax.fori_loop(0, (count + _L - 1) // _L, collect, 0)

                def extract(h, fs):
                    hp = h * _L
                    rel = hit_rel[pl.ds(jnp.minimum(hp, _CAP - _L), _L)]
                    bb = hit_b[pl.ds(jnp.minimum(hp, _CAP - _L), _L)]
                    valid = hp + lane < hcount
                    nv = plsc.all_reduce_population_count(valid)[0]
                    slots = fs + lane
                    for d in range(dim):
                        comp = plsc.load_gather(blkbuf, [zeros + d, rel],
                                                mask=valid)
                        plsc.store_scatter(rowflight, [slots, zeros + d],
                                           comp, mask=valid)
                    plsc.store_scatter(flight_b, [slots],
                                       jnp.where(valid, bb, neg1), mask=valid)
                    fs = fs + nv

                    @pl.when(fs > _FCAP - _L)
                    def _():
                        pltpu.async_copy(
                            rowflight,
                            stage_hbm.at[plsc.Indices(flight_b,
                                                      ignored_value=-1)],
                            ssem).wait()
                        reset_flight_b()

                    return jnp.where(fs > _FCAP - _L, 0, fs)

                return lax.fori_loop(
                    0, (hcount + _L - 1) // _L, extract, fslot)

            startb(0, blk0, sem0)

            def pair_body(p, fslot):
                startb(2 * p + 1, blk1, sem1)
                waitb(blk0, sem0)
                fslot = process(jnp.minimum(2 * p, nblk - 1), blk0, fslot)
                startb(2 * p + 2, blk0, sem0)
                waitb(blk1, sem1)
                fslot = process(jnp.minimum(2 * p + 1, nblk - 1), blk1, fslot)
                return fslot

            fslot = lax.fori_loop(0, (nblk + 1) // 2, pair_body, 0)
            waitb(blk0, sem0)

            @pl.when(fslot > 0)
            def _():
                pltpu.async_copy(
                    rowflight,
                    stage_hbm.at[plsc.Indices(flight_b, ignored_value=-1)],
                    ssem).wait()

        run_side(uidx_hbm, uembt_hbm, stage_u_hbm, nblk_u, nu)
        run_side(midx_hbm, membt_hbm, stage_m_hbm, nblk_m, nm)

    return k


@functools.lru_cache(maxsize=None)
def _make_phase2(batch: int, dim: int):
    mesh = plsc.VectorSubcoreMesh(core_axis_name="c", subcore_axis_name="s")
    b_per_w = batch // _NW
    chunk = 128
    n_chunks = b_per_w // chunk

    @functools.partial(
        pl.kernel,
        mesh=mesh,
        compiler_params=pltpu.CompilerParams(needs_layout_passes=False),
        out_type=jax.ShapeDtypeStruct((batch,), jnp.float32),
        scratch_types=[
            pltpu.VMEM((chunk, _STAGE_W), jnp.float32),
            pltpu.VMEM((chunk, _STAGE_W), jnp.float32),
            pltpu.VMEM((b_per_w,), jnp.int32),
            pltpu.VMEM((b_per_w,), jnp.int32),
            pltpu.VMEM((b_per_w,), jnp.float32),
            pltpu.VMEM((b_per_w,), jnp.float32),
            pltpu.VMEM((b_per_w,), jnp.float32),
            pltpu.SemaphoreType.DMA,
            pltpu.SemaphoreType.DMA,
        ],
    )
    def k(stage_u_hbm, stage_m_hbm, uidx_hbm, midx_hbm, ub_hbm, mb_hbm,
          out_hbm, cu, cm, uix, mix, ubv, mbv, out_v, sem, bsem):
        w = lax.axis_index("s") * 2 + lax.axis_index("c")
        base = w * b_per_w
        lane = lax.iota(jnp.int32, _L)
        zeros = jnp.zeros((_L,), jnp.int32)

        pltpu.sync_copy(uidx_hbm.at[pl.ds(base, b_per_w)], uix)
        pltpu.sync_copy(midx_hbm.at[pl.ds(base, b_per_w)], mix)
        pltpu.async_copy(ub_hbm.at[uix], ubv, bsem).wait()
        pltpu.async_copy(mb_hbm.at[mix], mbv, bsem).wait()

        def chunk_body(c, carry):
            r0 = base + c * chunk
            pltpu.async_copy(stage_u_hbm.at[pl.ds(r0, chunk), :], cu, sem)
            pltpu.async_copy(stage_m_hbm.at[pl.ds(r0, chunk), :], cm, sem)
            pltpu.make_async_copy(stage_u_hbm.at[pl.ds(r0, chunk), :], cu,
                                  sem).wait()
            pltpu.make_async_copy(stage_m_hbm.at[pl.ds(r0, chunk), :], cm,
                                  sem).wait()
            o0 = c * chunk

            def group(g, gcarry):
                b0 = g * _L
                out_v[pl.ds(o0 + b0, _L)] = (ubv[pl.ds(o0 + b0, _L)]
                                             + mbv[pl.ds(o0 + b0, _L)])
                for r in range(_L):
                    b = b0 + r
                    t = (cu[b, pl.ds(0, _L)] * cm[b, pl.ds(0, _L)]
                         + cu[b, pl.ds(_L, _L)] * cm[b, pl.ds(_L, _L)])
                    plsc.addupdate_scatter(out_v, [zeros + o0 + b], t)
                return gcarry

            lax.fori_loop(0, chunk // _L, group, 0)
            return carry

        lax.fori_loop(0, n_chunks, chunk_body, 0)
        pltpu.sync_copy(out_v, out_hbm.at[pl.ds(base, b_per_w)])

    return k


def kernel(user_indices, movie_indices, user_emb, movie_emb, user_bias, movie_bias):
    batch = user_indices.shape[0]
    nu, dim = user_emb.shape
    nm = movie_emb.shape[0]
    p1 = _make_phase1(batch, dim, nu, nm)
    p2 = _make_phase2(batch, dim)
    uidx = user_indices.astype(jnp.int32)
    midx = movie_indices.astype(jnp.int32)
    stage_u, stage_m = p1(uidx, midx, user_emb.T, movie_emb.T)
    return p2(stage_u, stage_m, uidx, midx,
              user_bias.reshape(-1), movie_bias.reshape(-1))


# speedup vs baseline: 3.9530x; 1.0507x over previous
import functools

import jax
import jax.numpy as jnp
from jax import lax
from jax.experimental import pallas as pl
from jax.experimental.pallas import tpu as pltpu
from jax.experimental.pallas import tpu_sc as plsc

_L = 16
_NW = 32
_BLK = 1024
_CAP = 2048
_FCAP = 128
_STAGE_W = 128


def _owned_blocks(w, nblk_total):
    base = nblk_total // _NW
    extra = nblk_total % _NW
    nblk = base + jnp.where(w < extra, 1, 0)
    blk_lo = w * base + jnp.minimum(w, extra)
    return blk_lo, nblk


@functools.lru_cache(maxsize=None)
def _make_phase1(batch: int, dim: int, nu: int, nm: int):
    mesh = plsc.VectorSubcoreMesh(core_axis_name="c", subcore_axis_name="s")
    nblk_u = -(-nu // _BLK)
    nblk_m = -(-nm // _BLK)

    @functools.partial(
        pl.kernel,
        mesh=mesh,
        compiler_params=pltpu.CompilerParams(needs_layout_passes=False),
        out_type=(
            jax.ShapeDtypeStruct((batch, _STAGE_W), jnp.float32),
            jax.ShapeDtypeStruct((batch, _STAGE_W), jnp.float32),
        ),
        scratch_types=[
            pltpu.VMEM((batch,), jnp.int32),
            pltpu.VMEM((batch,), jnp.int32),
            pltpu.VMEM((_CAP,), jnp.int32),
            pltpu.VMEM((_CAP,), jnp.int32),
            pltpu.VMEM((_CAP,), jnp.int32),
            pltpu.VMEM((_CAP,), jnp.int32),
            pltpu.VMEM((_CAP,), jnp.int32),
            pltpu.VMEM((_CAP,), jnp.int32),
            pltpu.VMEM((dim, _BLK), jnp.float32),
            pltpu.VMEM((dim, _BLK), jnp.float32),
            pltpu.VMEM((_FCAP, _STAGE_W), jnp.float32),
            pltpu.VMEM((_FCAP,), jnp.int32),
            pltpu.SemaphoreType.DMA,
            pltpu.SemaphoreType.DMA,
            pltpu.SemaphoreType.DMA,
        ],
    )
    def k(uidx_hbm, midx_hbm, uembt_hbm, membt_hbm,
          stage_u_hbm, stage_m_hbm,
          uidx_v, midx_v, uown_rel, uown_b, mown_rel, mown_b,
          hit_rel, hit_b, blk0, blk1,
          rowflight, flight_b, sem0, sem1, ssem):
        w = lax.axis_index("s") * 2 + lax.axis_index("c")
        lane = lax.iota(jnp.int32, _L)
        zeros = jnp.zeros((_L,), jnp.int32)
        neg1 = zeros - 1

        def reset_flight_b():
            for q in range(_FCAP // _L):
                flight_b[pl.ds(q * _L, _L)] = neg1

        ublk_lo, unblk = _owned_blocks(w, nblk_u)
        mblk_lo, mnblk = _owned_blocks(w, nblk_m)
        pltpu.sync_copy(uidx_hbm, uidx_v)
        pltpu.sync_copy(midx_hbm, midx_v)
        ucol0 = pl.multiple_of(ublk_lo * _BLK, _BLK)
        pltpu.async_copy(uembt_hbm.at[:, pl.ds(ucol0, _BLK)], blk0, sem0)

        urow_lo = ublk_lo * _BLK
        mrow_lo = mblk_lo * _BLK
        unrows = unblk * _BLK
        mnrows = mnblk * _BLK

        def sel(g, carry):
            uptr, mptr = carry
            bvec = lane + g * _L
            uv = uidx_v[pl.ds(g * _L, _L)]
            urel = uv - urow_lo
            umsk = (urel >= 0) & (urel < unrows)
            up = jnp.minimum(uptr, _CAP - _L)
            plsc.store_compressed(uown_rel.at[pl.ds(up, _L)], urel, mask=umsk)
            plsc.store_compressed(uown_b.at[pl.ds(up, _L)], bvec, mask=umsk)
            mv = midx_v[pl.ds(g * _L, _L)]
            mrel = mv - mrow_lo
            mmsk = (mrel >= 0) & (mrel < mnrows)
            mp = jnp.minimum(mptr, _CAP - _L)
            plsc.store_compressed(mown_rel.at[pl.ds(mp, _L)], mrel, mask=mmsk)
            plsc.store_compressed(mown_b.at[pl.ds(mp, _L)], bvec, mask=mmsk)
            ucnt = plsc.all_reduce_population_count(umsk)
            mcnt = plsc.all_reduce_population_count(mmsk)
            return uptr + ucnt[0], mptr + mcnt[0]

        ucount, mcount = lax.fori_loop(0, batch // _L, sel, (0, 0))
        ucount = jnp.minimum(ucount, _CAP)
        mcount = jnp.minimum(mcount, _CAP)

        def run_side(embt_hbm, stage_hbm, blk_lo, nblk, own_rel, own_b,
                     count, primed):
            reset_flight_b()

            def startb(blk, blkbuf, s):
                bb = jnp.minimum(blk, nblk - 1)
                col0 = pl.multiple_of((blk_lo + bb) * _BLK, _BLK)
                pltpu.async_copy(embt_hbm.at[:, pl.ds(col0, _BLK)], blkbuf, s)

            def waitb(blkbuf, s):
                pltpu.make_async_copy(
                    embt_hbm.at[:, pl.ds(0, _BLK)], blkbuf, s).wait()

            def process(blk, blkbuf, fslot):
                lo = blk * _BLK

                def collect(g, hptr):
                    gp = g * _L
                    rel = own_rel[pl.ds(jnp.minimum(gp, _CAP - _L), _L)]
                    bb = own_b[pl.ds(jnp.minimum(gp, _CAP - _L), _L)]
                    msk = ((gp + lane < count)
                           & (rel >= lo) & (rel < lo + _BLK))
                    p = jnp.minimum(hptr, _CAP - _L)
                    plsc.store_compressed(hit_rel.at[pl.ds(p, _L)],
                                          rel - lo, mask=msk)
                    plsc.store_compressed(hit_b.at[pl.ds(p, _L)], bb, mask=msk)
                    cnt = plsc.all_reduce_population_count(msk)
                    return hptr + cnt[0]

                hcount = lax.fori_loop(0, (count + _L - 1) // _L, collect, 0)

                def extract(h, fs):
                    hp = h * _L
                    rel = hit_rel[pl.ds(jnp.minimum(hp, _CAP - _L), _L)]
                    bb = hit_b[pl.ds(jnp.minimum(hp, _CAP - _L), _L)]
                    valid = hp + lane < hcount
                    nv = plsc.all_reduce_population_count(valid)[0]
                    slots = fs + lane
                    for d in range(dim):
                        comp = plsc.load_gather(blkbuf, [zeros + d, rel],
                                                mask=valid)
                        plsc.store_scatter(rowflight, [slots, zeros + d],
                                           comp, mask=valid)
                    plsc.store_scatter(flight_b, [slots],
                                       jnp.where(valid, bb, neg1), mask=valid)
                    fs = fs + nv

                    @pl.when(fs > _FCAP - _L)
                    def _():
                        pltpu.async_copy(
                            rowflight,
                            stage_hbm.at[plsc.Indices(flight_b,
                                                      ignored_value=-1)],
                            ssem).wait()
                        reset_flight_b()

                    return jnp.where(fs > _FCAP - _L, 0, fs)

                return lax.fori_loop(
                    0, (hcount + _L - 1) // _L, extract, fslot)

            if not primed:
                startb(0, blk0, sem0)

            def pair_body(p, fslot):
                startb(2 * p + 1, blk1, sem1)
                waitb(blk0, sem0)
                fslot = process(jnp.minimum(2 * p, nblk - 1), blk0, fslot)
                startb(2 * p + 2, blk0, sem0)
                waitb(blk1, sem1)
                fslot = process(jnp.minimum(2 * p + 1, nblk - 1), blk1, fslot)
                return fslot

            fslot = lax.fori_loop(0, (nblk + 1) // 2, pair_body, 0)
            waitb(blk0, sem0)

            @pl.when(fslot > 0)
            def _():
                pltpu.async_copy(
                    rowflight,
                    stage_hbm.at[plsc.Indices(flight_b, ignored_value=-1)],
                    ssem).wait()

        run_side(uembt_hbm, stage_u_hbm, ublk_lo, unblk, uown_rel, uown_b,
                 ucount, True)
        run_side(membt_hbm, stage_m_hbm, mblk_lo, mnblk, mown_rel, mown_b,
                 mcount, False)

    return k


@functools.lru_cache(maxsize=None)
def _make_phase2(batch: int, dim: int):
    mesh = plsc.VectorSubcoreMesh(core_axis_name="c", subcore_axis_name="s")
    b_per_w = batch // _NW
    chunk = 128
    n_chunks = b_per_w // chunk

    @functools.partial(
        pl.kernel,
        mesh=mesh,
        compiler_params=pltpu.CompilerParams(needs_layout_passes=False),
        out_type=jax.ShapeDtypeStruct((batch,), jnp.float32),
        scratch_types=[
            pltpu.VMEM((chunk, _STAGE_W), jnp.float32),
            pltpu.VMEM((chunk, _STAGE_W), jnp.float32),
            pltpu.VMEM((chunk, _STAGE_W), jnp.float32),
            pltpu.VMEM((chunk, _STAGE_W), jnp.float32),
            pltpu.VMEM((b_per_w,), jnp.int32),
            pltpu.VMEM((b_per_w,), jnp.int32),
            pltpu.VMEM((b_per_w,), jnp.float32),
            pltpu.VMEM((b_per_w,), jnp.float32),
            pltpu.VMEM((b_per_w,), jnp.float32),
            pltpu.SemaphoreType.DMA,
            pltpu.SemaphoreType.DMA,
            pltpu.SemaphoreType.DMA,
        ],
    )
    def k(stage_u_hbm, stage_m_hbm, uidx_hbm, midx_hbm, ub_hbm, mb_hbm,
          out_hbm, cu0, cm0, cu1, cm1, uix, mix, ubv, mbv, out_v,
          sem0, sem1, bsem):
        w = lax.axis_index("s") * 2 + lax.axis_index("c")
        base = w * b_per_w
        lane = lax.iota(jnp.int32, _L)
        zeros = jnp.zeros((_L,), jnp.int32)

        def startc(c, bu, bm, s):
            r0 = base + jnp.minimum(c, n_chunks - 1) * chunk
            pltpu.async_copy(stage_u_hbm.at[pl.ds(r0, chunk), :], bu, s)
            pltpu.async_copy(stage_m_hbm.at[pl.ds(r0, chunk), :], bm, s)

        def waitc(bu, bm, s):
            pltpu.make_async_copy(stage_u_hbm.at[pl.ds(0, chunk), :], bu,
                                  s).wait()
            pltpu.make_async_copy(stage_m_hbm.at[pl.ds(0, chunk), :], bm,
                                  s).wait()

        startc(0, cu0, cm0, sem0)
        pltpu.sync_copy(uidx_hbm.at[pl.ds(base, b_per_w)], uix)
        pltpu.sync_copy(midx_hbm.at[pl.ds(base, b_per_w)], mix)
        pltpu.async_copy(ub_hbm.at[uix], ubv, bsem).wait()
        pltpu.async_copy(mb_hbm.at[mix], mbv, bsem).wait()

        def proc(c, bu, bm):
            o0 = jnp.minimum(c, n_chunks - 1) * chunk

            def group(g, gcarry):
                b0 = g * _L
                out_v[pl.ds(o0 + b0, _L)] = (ubv[pl.ds(o0 + b0, _L)]
                                             + mbv[pl.ds(o0 + b0, _L)])
                for r in range(_L):
                    b = b0 + r
                    t = (bu[b, pl.ds(0, _L)] * bm[b, pl.ds(0, _L)]
                         + bu[b, pl.ds(_L, _L)] * bm[b, pl.ds(_L, _L)])
                    plsc.addupdate_scatter(out_v, [zeros + o0 + b], t)
                return gcarry

            lax.fori_loop(0, chunk // _L, group, 0)

        def pair_body(p, carry):
            startc(2 * p + 1, cu1, cm1, sem1)
            waitc(cu0, cm0, sem0)
            proc(2 * p, cu0, cm0)
            startc(2 * p + 2, cu0, cm0, sem0)
            waitc(cu1, cm1, sem1)
            proc(2 * p + 1, cu1, cm1)
            return carry

        lax.fori_loop(0, (n_chunks + 1) // 2, pair_body, 0)
        waitc(cu0, cm0, sem0)
        pltpu.sync_copy(out_v, out_hbm.at[pl.ds(base, b_per_w)])

    return k


def kernel(user_indices, movie_indices, user_emb, movie_emb, user_bias, movie_bias):
    batch = user_indices.shape[0]
    nu, dim = user_emb.shape
    nm = movie_emb.shape[0]
    p1 = _make_phase1(batch, dim, nu, nm)
    p2 = _make_phase2(batch, dim)
    uidx = user_indices.astype(jnp.int32)
    midx = movie_indices.astype(jnp.int32)
    stage_u, stage_m = p1(uidx, midx, user_emb.T, movie_emb.T)
    return p2(stage_u, stage_m, uidx, midx,
              user_bias.reshape(-1), movie_bias.reshape(-1))
